# baseline (speedup 1.0000x reference)
import functools

import jax
import jax.numpy as jnp
from jax import lax
from jax.experimental import pallas as pl
from jax.experimental.pallas import tpu as pltpu
from jax.experimental.pallas import tpu_sc as plsc

LAYERS = 10
EMB = 20
EMBP = 32
LEN = (4 ** LAYERS - 1) // 3
BATCH = 4096
SEL = 50

_THRESH = [(4 ** l - 1) // 3 for l in range(1, LAYERS)]

_NC, _NS = 2, 16
_NW = _NC * _NS
_FLAT = BATCH * SEL
_R16 = (LEN * EMB + 12) // 16
_CHUNK = 128
_RPC = _CHUNK // 2
_NCH = _FLAT // _RPC // _NW
_GRP = 10
_NGRP = _NCH // _GRP


def _sc_gather(t16, idx3d):
    mesh = plsc.VectorSubcoreMesh(core_axis_name="c", subcore_axis_name="s")

    @functools.partial(
        pl.kernel,
        mesh=mesh,
        out_type=jax.ShapeDtypeStruct((_FLAT // _RPC, _CHUNK, 16), jnp.float32),
        scratch_types=[
            pltpu.VMEM((_NCH, _CHUNK), jnp.int32),
            pltpu.VMEM((2, _GRP, _CHUNK, 16), jnp.float32),
            pltpu.SemaphoreType.DMA,
            pltpu.SemaphoreType.DMA,
        ],
        compiler_params=pltpu.CompilerParams(use_tc_tiling_on_sc=False),
    )
    def k(t16_hbm, idx_hbm, out_hbm, idx_v, rows_v, sem0, sem1):
        wid = lax.axis_index("s") * _NC + lax.axis_index("c")
        row0 = wid * _NCH
        pltpu.sync_copy(idx_hbm.at[wid], idx_v)
        sems = [sem0, sem1]
        handles = [None, None]

        def fire(g):
            b = g % 2
            handles[b] = [
                pltpu.async_copy(
                    t16_hbm.at[idx_v.at[g * _GRP + j]], rows_v.at[b].at[j], sems[b]
                )
                for j in range(_GRP)
            ]

        fire(0)
        for g in range(_NGRP):
            if g + 1 < _NGRP:
                fire(g + 1)
            for h in handles[g % 2]:
                h.wait()
            pltpu.sync_copy(rows_v.at[g % 2], out_hbm.at[pl.ds(row0 + g * _GRP, _GRP)])

    return k(t16, idx3d)


_RB = 256
_GW = SEL * EMBP


def _reduce_body(g_ref, idx_ref, na_ref):
    idx = idx_ref[...]
    c = jnp.full(idx.shape, 1.0, jnp.float32)
    for t in _THRESH:
        c = jnp.where(idx >= t, c * 0.25, c)
    off = 4 * (idx % 4)
    g = g_ref[...]
    li = lax.broadcasted_iota(jnp.int32, (SEL, _GW), 1) // EMBP
    si = lax.broadcasted_iota(jnp.int32, (SEL, _GW), 0)
    e = (li == si).astype(jnp.float32)
    a = jnp.zeros((_RB, EMB), jnp.float32)
    for o in (0, 4, 8, 12):
        co = jnp.where(off == o, c, 0.0)
        m = lax.dot_general(co, e, (((1,), (0,)), ((), ())),
                            preferred_element_type=jnp.float32)
        x = g * m
        h = x[:, :800] + x[:, 800:]
        y5 = jnp.zeros((_RB, 160), jnp.float32)
        for k in range(5):
            y5 = y5 + h[:, 160 * k:160 * (k + 1)]
        y = jnp.zeros((_RB, EMBP), jnp.float32)
        for k in range(5):
            y = y + y5[:, EMBP * k:EMBP * (k + 1)]
        a = a + y[:, o:o + EMB]
    norm = jnp.sum(jnp.abs(a), axis=1, keepdims=True)
    na = a / norm
    na_ref[...] = jnp.concatenate(
        [na, jnp.zeros((_RB, EMBP - EMB), jnp.float32)], axis=1)


def _tc_reduce(g_flat, indices):
    return pl.pallas_call(
        _reduce_body,
        grid=(BATCH // _RB,),
        in_specs=[
            pl.BlockSpec((_RB, _GW), lambda i: (i, 0)),
            pl.BlockSpec((_RB, SEL), lambda i: (i, 0)),
        ],
        out_specs=pl.BlockSpec((_RB, EMBP), lambda i: (i, 0)),
        out_shape=jax.ShapeDtypeStruct((BATCH, EMBP), jnp.float32),
    )(g_flat, indices)


_JB = 512


def _colsum_body(na_ref, naj_ref, cs_ref):
    m = lax.dot_general(
        na_ref[...], naj_ref[...], (((1,), (1,)), ((), ())),
        preferred_element_type=jnp.float32,
    )
    cs_ref[...] = jnp.sum(jnp.abs(m), axis=0, keepdims=True)


def _tc_colsum(na):
    return pl.pallas_call(
        _colsum_body,
        grid=(BATCH // _JB,),
        in_specs=[
            pl.BlockSpec((BATCH, EMBP), lambda j: (0, 0)),
            pl.BlockSpec((_JB, EMBP), lambda j: (j, 0)),
        ],
        out_specs=pl.BlockSpec((1, _JB), lambda j: (0, j)),
        out_shape=jax.ShapeDtypeStruct((1, BATCH), jnp.float32),
    )(na, na)


def _final_body(nai_ref, naj_ref, cs_ref, out_ref):
    m = lax.dot_general(
        nai_ref[...], naj_ref[...], (((1,), (1,)), ((), ())),
        preferred_element_type=jnp.float32,
    )
    out_ref[...] = m / cs_ref[...]


def _tc_final(na, cs):
    return pl.pallas_call(
        _final_body,
        grid=(BATCH // _JB, BATCH // _JB),
        in_specs=[
            pl.BlockSpec((_JB, EMBP), lambda i, j: (i, 0)),
            pl.BlockSpec((_JB, EMBP), lambda i, j: (j, 0)),
            pl.BlockSpec((1, _JB), lambda i, j: (0, j)),
        ],
        out_specs=pl.BlockSpec((_JB, _JB), lambda i, j: (i, j)),
        out_shape=jax.ShapeDtypeStruct((BATCH, BATCH), jnp.float32),
    )(na, na, cs)


def kernel(nodes_table, indices, layers_arr):
    del layers_arr
    t16 = jnp.pad(nodes_table.reshape(-1), (0, _R16 * 16 - LEN * EMB)).reshape(_R16, 16)
    i = indices.reshape(-1)
    r0 = (5 * i) // 4
    il = jnp.stack([r0, r0 + 1], axis=-1).reshape(_NW, _NCH, _CHUNK)
    gathered = _sc_gather(t16, il)
    g_flat = gathered.reshape(BATCH, _GW)
    na = _tc_reduce(g_flat, indices)
    cs = _tc_colsum(na)
    return _tc_final(na, cs)


# speedup vs baseline: 1.0595x; 1.0595x over previous
import functools

import jax
import jax.numpy as jnp
from jax import lax
from jax.experimental import pallas as pl
from jax.experimental.pallas import tpu as pltpu
from jax.experimental.pallas import tpu_sc as plsc

LAYERS = 10
EMB = 20
EMBP = 32
LEN = (4 ** LAYERS - 1) // 3
BATCH = 4096
SEL = 50

_THRESH = [(4 ** l - 1) // 3 for l in range(1, LAYERS)]

_NC, _NS = 2, 16
_NW = _NC * _NS
_FLAT = BATCH * SEL
_R16 = (LEN * EMB + 12) // 16
_CHUNK = 128
_RPC = _CHUNK // 2
_NCH = _FLAT // _RPC // _NW
_GRP = 10
_NGRP = _NCH // _GRP


def _sc_gather(t16, idx3d):
    mesh = plsc.VectorSubcoreMesh(core_axis_name="c", subcore_axis_name="s")

    @functools.partial(
        pl.kernel,
        mesh=mesh,
        out_type=jax.ShapeDtypeStruct((_FLAT // _RPC, _CHUNK, 16), jnp.float32),
        scratch_types=[
            pltpu.VMEM((_NCH, _CHUNK), jnp.int32),
            pltpu.VMEM((2, _GRP, _CHUNK, 16), jnp.float32),
            pltpu.SemaphoreType.DMA,
            pltpu.SemaphoreType.DMA,
        ],
        compiler_params=pltpu.CompilerParams(use_tc_tiling_on_sc=False),
    )
    def k(t16_hbm, idx_hbm, out_hbm, idx_v, rows_v, sem0, sem1):
        wid = lax.axis_index("s") * _NC + lax.axis_index("c")
        row0 = wid * _NCH
        pltpu.sync_copy(idx_hbm.at[wid], idx_v)
        sems = [sem0, sem1]
        handles = [None, None]

        def fire(g):
            b = g % 2
            handles[b] = [
                pltpu.async_copy(
                    t16_hbm.at[idx_v.at[g * _GRP + j]], rows_v.at[b].at[j], sems[b]
                )
                for j in range(_GRP)
            ]

        fire(0)
        for g in range(_NGRP):
            if g + 1 < _NGRP:
                fire(g + 1)
            for h in handles[g % 2]:
                h.wait()
            pltpu.sync_copy(rows_v.at[g % 2], out_hbm.at[pl.ds(row0 + g * _GRP, _GRP)])

    return k(t16, idx3d)


_RB = 256
_GW = SEL * EMBP


def _reduce_body(g_ref, idx_ref, na_ref):
    idx = idx_ref[...]
    c = jnp.full(idx.shape, 1.0, jnp.float32)
    for t in _THRESH:
        c = jnp.where(idx >= t, c * 0.25, c)
    off = 4 * (idx % 4)
    g = g_ref[...]
    li = lax.broadcasted_iota(jnp.int32, (SEL, _GW), 1) // EMBP
    si = lax.broadcasted_iota(jnp.int32, (SEL, _GW), 0)
    e = (li == si).astype(jnp.float32)
    a = jnp.zeros((_RB, EMB), jnp.float32)
    for o in (0, 4, 8, 12):
        co = jnp.where(off == o, c, 0.0)
        m = lax.dot_general(co, e, (((1,), (0,)), ((), ())),
                            preferred_element_type=jnp.float32)
        x = g * m
        h = x[:, :800] + x[:, 800:]
        y5 = jnp.zeros((_RB, 160), jnp.float32)
        for k in range(5):
            y5 = y5 + h[:, 160 * k:160 * (k + 1)]
        y = jnp.zeros((_RB, EMBP), jnp.float32)
        for k in range(5):
            y = y + y5[:, EMBP * k:EMBP * (k + 1)]
        a = a + y[:, o:o + EMB]
    norm = jnp.sum(jnp.abs(a), axis=1, keepdims=True)
    na = a / norm
    na_ref[...] = jnp.concatenate(
        [na, jnp.zeros((_RB, EMBP - EMB), jnp.float32)], axis=1)


def _tc_reduce(g_flat, indices):
    return pl.pallas_call(
        _reduce_body,
        grid=(BATCH // _RB,),
        in_specs=[
            pl.BlockSpec((_RB, _GW), lambda i: (i, 0)),
            pl.BlockSpec((_RB, SEL), lambda i: (i, 0)),
        ],
        out_specs=pl.BlockSpec((_RB, EMBP), lambda i: (i, 0)),
        out_shape=jax.ShapeDtypeStruct((BATCH, EMBP), jnp.float32),
    )(g_flat, indices)


_JB = 512


def _colsum_body(na_ref, naj_ref, cs_ref):
    m = lax.dot_general(
        na_ref[...], naj_ref[...], (((1,), (1,)), ((), ())),
        preferred_element_type=jnp.float32,
    )
    cs_ref[...] = jnp.sum(jnp.abs(m), axis=0, keepdims=True)


def _tc_colsum(na):
    return pl.pallas_call(
        _colsum_body,
        grid=(BATCH // _JB,),
        in_specs=[
            pl.BlockSpec((BATCH, EMBP), lambda j: (0, 0)),
            pl.BlockSpec((_JB, EMBP), lambda j: (j, 0)),
        ],
        out_specs=pl.BlockSpec((1, _JB), lambda j: (0, j)),
        out_shape=jax.ShapeDtypeStruct((1, BATCH), jnp.float32),
    )(na, na)


def _final_body(nai_ref, naj_ref, cs_ref, out_ref):
    m = lax.dot_general(
        nai_ref[...], naj_ref[...], (((1,), (1,)), ((), ())),
        preferred_element_type=jnp.float32,
    )
    out_ref[...] = m / cs_ref[...]


_FB = 1024


def _tc_final(na, cs):
    return pl.pallas_call(
        _final_body,
        grid=(BATCH // _FB, BATCH // _FB),
        in_specs=[
            pl.BlockSpec((_FB, EMBP), lambda i, j: (i, 0)),
            pl.BlockSpec((_FB, EMBP), lambda i, j: (j, 0)),
            pl.BlockSpec((1, _FB), lambda i, j: (0, j)),
        ],
        out_specs=pl.BlockSpec((_FB, _FB), lambda i, j: (i, j)),
        out_shape=jax.ShapeDtypeStruct((BATCH, BATCH), jnp.float32),
    )(na, na, cs)


def kernel(nodes_table, indices, layers_arr):
    del layers_arr
    t16 = jnp.pad(nodes_table.reshape(-1), (0, _R16 * 16 - LEN * EMB)).reshape(_R16, 16)
    i = indices.reshape(-1)
    r0 = (5 * i) // 4
    il = jnp.stack([r0, r0 + 1], axis=-1).reshape(_NW, _NCH, _CHUNK)
    gathered = _sc_gather(t16, il)
    g_flat = gathered.reshape(BATCH, _GW)
    na = _tc_reduce(g_flat, indices)
    cs = _tc_colsum(na)
    return _tc_final(na, cs)


# speedup vs baseline: 1.0632x; 1.0034x over previous
import functools

import jax
import jax.numpy as jnp
from jax import lax
from jax.experimental import pallas as pl
from jax.experimental.pallas import tpu as pltpu
from jax.experimental.pallas import tpu_sc as plsc

LAYERS = 10
EMB = 20
EMBP = 32
LEN = (4 ** LAYERS - 1) // 3
BATCH = 4096
SEL = 50

_THRESH = [(4 ** l - 1) // 3 for l in range(1, LAYERS)]

_NC, _NS = 2, 16
_NW = _NC * _NS

_RCH = 1024
_NFULL = LEN // _RCH
_LASTN = LEN - _NFULL * _RCH
_SLOTS = (_NFULL + _NW - 1) // _NW
_T32R = (LEN * EMBP + 127) // 128
_FLATW = _T32R * 128


def _sc_relayout(table_t):
    mesh = plsc.VectorSubcoreMesh(core_axis_name="c", subcore_axis_name="s")

    @functools.partial(
        pl.kernel,
        mesh=mesh,
        out_type=jax.ShapeDtypeStruct((_FLATW,), jnp.float32),
        scratch_types=[
            pltpu.VMEM((EMB, _RCH), jnp.float32),
            pltpu.VMEM((_RCH * EMBP,), jnp.float32),
        ],
        compiler_params=pltpu.CompilerParams(needs_layout_passes=False),
    )
    def k(tt_hbm, out_hbm, inbuf, stg):
        wid = lax.axis_index("s") * _NC + lax.axis_index("c")
        iota = lax.iota(jnp.int32, 16)
        i32 = iota * EMBP

        def do_chunk(c, nrows):
            ngrp = (nrows + 15) // 16
            for g in range(ngrp):
                base = i32 + (g * 16 * EMBP)
                if nrows - g * 16 >= 16:
                    msk = None
                else:
                    msk = iota < (nrows - g * 16)
                for d in range(EMB):
                    v = inbuf[d, pl.ds(16 * g, 16)]
                    plsc.store_scatter(stg, [base + d], v, mask=msk)
            pltpu.sync_copy(stg.at[pl.ds(0, ((nrows * EMBP + 7) // 8) * 8)],
                            out_hbm.at[pl.ds(c * _RCH * EMBP,
                                             ((nrows * EMBP + 7) // 8) * 8)])

        def body(j, carry):
            c = wid * _SLOTS + j

            @pl.when(c < _NFULL)
            def _():
                pltpu.sync_copy(tt_hbm.at[:, pl.ds(c * _RCH, _RCH)], inbuf)
                do_chunk(c, _RCH)

            return carry

        lax.fori_loop(0, _SLOTS, body, 0)

        @pl.when(wid == _NW - 1)
        def _():
            pltpu.sync_copy(tt_hbm.at[:, pl.ds(_NFULL * _RCH, 384)],
                            inbuf.at[:, pl.ds(0, 384)])
            do_chunk(_NFULL, _LASTN)

    return k(table_t)


_FLAT = BATCH * SEL
_RPW = _FLAT // _NW
_GCH = 128
_NCH = _RPW // _GCH
_BPW = BATCH // _NW


def _sc_gather_reduce(t32, i3):
    mesh = plsc.VectorSubcoreMesh(core_axis_name="c", subcore_axis_name="s")

    @functools.partial(
        pl.kernel,
        mesh=mesh,
        out_type=jax.ShapeDtypeStruct((BATCH, EMBP), jnp.float32),
        scratch_types=[
            pltpu.VMEM((_NCH, _GCH), jnp.int32),
            pltpu.VMEM((_NCH, _GCH), jnp.int32),
            pltpu.VMEM((_NCH, _GCH), jnp.float32),
            pltpu.VMEM((2, _GCH, 128), jnp.float32),
            pltpu.VMEM((_BPW, EMBP), jnp.float32),
            pltpu.SemaphoreType.DMA,
            pltpu.SemaphoreType.DMA,
        ],
        compiler_params=pltpu.CompilerParams(needs_layout_passes=False),
    )
    def k(t32_hbm, i3_hbm, out_hbm, ibuf, qbuf, cbuf, dst, acc, sem0, sem1):
        wid = lax.axis_index("s") * _NC + lax.axis_index("c")
        iota = lax.iota(jnp.int32, 16)
        zero = jnp.zeros((16,), jnp.float32)
        pltpu.sync_copy(i3_hbm.at[wid], ibuf)
        sems = [sem0, sem1]

        def prep(r, carry):
            for g in range(8):
                iv = ibuf[r, pl.ds(16 * g, 16)]
                qbuf[r, pl.ds(16 * g, 16)] = iv >> 2
                cf = jnp.full((16,), 1.0, jnp.float32)
                for t in _THRESH:
                    cf = jnp.where(iv >= t, cf * 0.25, cf)
                cbuf[r, pl.ds(16 * g, 16)] = cf
            return carry

        lax.fori_loop(0, _NCH, prep, 0)

        def zacc(b, carry):
            acc[b, pl.ds(0, 16)] = zero
            acc[b, pl.ds(16, 16)] = zero
            return carry

        lax.fori_loop(0, _BPW, zacc, 0)

        def fire(c, b):
            pltpu.async_copy(t32_hbm.at[qbuf.at[c]], dst.at[b], sems[b])

        def wait(b):
            pltpu.make_async_copy(t32_hbm.at[pl.ds(0, _GCH)], dst.at[b],
                                  sems[b]).wait()

        def process(c, b):
            for g in range(8):
                iv = ibuf[c, pl.ds(16 * g, 16)]
                cv = cbuf[c, pl.ds(16 * g, 16)]
                ov = (iv & 3) << 5
                rowv = iota + (g * 16)
                bv = (iota + (c * _GCH + g * 16)) // SEL
                for d in range(EMB):
                    v = plsc.load_gather(dst.at[b], [rowv, ov + d])
                    plsc.addupdate_scatter(acc, [bv, iota * 0 + d], cv * v)

        fire(0, 0)

        def loop(kk, carry):
            c0 = 2 * kk
            fire(c0 + 1, 1)
            wait(0)
            process(c0, 0)

            @pl.when(c0 + 2 < _NCH)
            def _():
                fire(c0 + 2, 0)

            wait(1)
            process(c0 + 1, 1)
            return carry

        lax.fori_loop(0, _NCH // 2, loop, 0)

        def norm(b, carry):
            a0 = acc[b, pl.ds(0, 16)]
            a1 = acc[b, pl.ds(16, 16)]
            n = jnp.sum(jnp.abs(a0), axis=0) + jnp.sum(jnp.abs(a1), axis=0)
            acc[b, pl.ds(0, 16)] = a0 / n
            acc[b, pl.ds(16, 16)] = a1 / n
            return carry

        lax.fori_loop(0, _BPW, norm, 0)
        pltpu.sync_copy(acc, out_hbm.at[pl.ds(wid * _BPW, _BPW)])

    return k(t32, i3)


_JB = 512
_FB = 1024


def _colsum_body(na_ref, naj_ref, cs_ref):
    m = lax.dot_general(
        na_ref[...], naj_ref[...], (((1,), (1,)), ((), ())),
        preferred_element_type=jnp.float32,
    )
    cs_ref[...] = jnp.sum(jnp.abs(m), axis=0, keepdims=True)


def _tc_colsum(na):
    return pl.pallas_call(
        _colsum_body,
        grid=(BATCH // _JB,),
        in_specs=[
            pl.BlockSpec((BATCH, EMBP), lambda j: (0, 0)),
            pl.BlockSpec((_JB, EMBP), lambda j: (j, 0)),
        ],
        out_specs=pl.BlockSpec((1, _JB), lambda j: (0, j)),
        out_shape=jax.ShapeDtypeStruct((1, BATCH), jnp.float32),
    )(na, na)


def _final_body(nai_ref, naj_ref, cs_ref, out_ref):
    m = lax.dot_general(
        nai_ref[...], naj_ref[...], (((1,), (1,)), ((), ())),
        preferred_element_type=jnp.float32,
    )
    out_ref[...] = m / cs_ref[...]


def _tc_final(na, cs):
    return pl.pallas_call(
        _final_body,
        grid=(BATCH // _FB, BATCH // _FB),
        in_specs=[
            pl.BlockSpec((_FB, EMBP), lambda i, j: (i, 0)),
            pl.BlockSpec((_FB, EMBP), lambda i, j: (j, 0)),
            pl.BlockSpec((1, _FB), lambda i, j: (0, j)),
        ],
        out_specs=pl.BlockSpec((_FB, _FB), lambda i, j: (i, j)),
        out_shape=jax.ShapeDtypeStruct((BATCH, BATCH), jnp.float32),
    )(na, na, cs)


def kernel(nodes_table, indices, layers_arr):
    del layers_arr
    ttp = jnp.pad(nodes_table.T, ((0, 0), (0, 349568 - LEN)))
    t32flat = _sc_relayout(ttp)
    t32 = t32flat.reshape(_T32R, 128)
    i3 = indices.reshape(_NW, _NCH, _GCH)
    na = _sc_gather_reduce(t32, i3)
    cs = _tc_colsum(na)
    return _tc_final(na, cs)


# speedup vs baseline: 1.0799x; 1.0157x over previous
import functools

import jax
import jax.numpy as jnp
from jax import lax
from jax.experimental import pallas as pl
from jax.experimental.pallas import tpu as pltpu
from jax.experimental.pallas import tpu_sc as plsc

LAYERS = 10
EMB = 20
EMBP = 32
LEN = (4 ** LAYERS - 1) // 3
BATCH = 4096
SEL = 50

_THRESH = [(4 ** l - 1) // 3 for l in range(1, LAYERS)]

_NC, _NS = 2, 16
_NW = _NC * _NS

_RCH = 1024
_NFULL = LEN // _RCH
_LASTN = LEN - _NFULL * _RCH
_SLOTS = (_NFULL + _NW - 1) // _NW
_T32R = (LEN * EMBP + 127) // 128
_FLATW = _T32R * 128


def _sc_relayout(table_t):
    mesh = plsc.VectorSubcoreMesh(core_axis_name="c", subcore_axis_name="s")

    @functools.partial(
        pl.kernel,
        mesh=mesh,
        out_type=jax.ShapeDtypeStruct((_FLATW,), jnp.float32),
        scratch_types=[
            pltpu.VMEM((2, EMB, _RCH), jnp.float32),
            pltpu.VMEM((_RCH * EMBP,), jnp.float32),
            pltpu.SemaphoreType.DMA,
            pltpu.SemaphoreType.DMA,
        ],
        compiler_params=pltpu.CompilerParams(needs_layout_passes=False),
    )
    def k(tt_hbm, out_hbm, inbuf, stg, isem0, isem1):
        wid = lax.axis_index("s") * _NC + lax.axis_index("c")
        iota = lax.iota(jnp.int32, 16)
        i32 = iota * EMBP
        isems = [isem0, isem1]
        ntot = _NFULL + 1

        def fire_in(c, h):
            @pl.when(c < _NFULL)
            def _():
                pltpu.async_copy(tt_hbm.at[:, pl.ds(c * _RCH, _RCH)],
                                 inbuf.at[h].at[:, pl.ds(0, _RCH)], isems[h])

            @pl.when(c == _NFULL)
            def _():
                pltpu.async_copy(tt_hbm.at[:, pl.ds(_NFULL * _RCH, 384)],
                                 inbuf.at[h].at[:, pl.ds(0, 384)], isems[h])

        def wait_in(c, h):
            @pl.when(c < _NFULL)
            def _():
                pltpu.make_async_copy(tt_hbm.at[:, pl.ds(0, _RCH)],
                                      inbuf.at[h].at[:, pl.ds(0, _RCH)],
                                      isems[h]).wait()

            @pl.when(c == _NFULL)
            def _():
                pltpu.make_async_copy(tt_hbm.at[:, pl.ds(0, 384)],
                                      inbuf.at[h].at[:, pl.ds(0, 384)],
                                      isems[h]).wait()

        def do_chunk(c, h):
            nv = jnp.where(c == _NFULL, _LASTN, _RCH)
            nvs = jnp.zeros((16,), jnp.int32) + nv
            for g in range(_RCH // 16):
                base = i32 + (g * 16 * EMBP)
                msk = (iota + (g * 16)) < nvs
                for d in range(EMB):
                    v = inbuf[h, d, pl.ds(16 * g, 16)]
                    plsc.store_scatter(stg, [base + d], v, mask=msk)
            @pl.when(c < _NFULL)
            def _():
                pltpu.sync_copy(stg,
                                out_hbm.at[pl.ds(c * _RCH * EMBP, _RCH * EMBP)])

            @pl.when(c == _NFULL)
            def _():
                pltpu.sync_copy(stg.at[pl.ds(0, _LASTN * EMBP)],
                                out_hbm.at[pl.ds(_NFULL * _RCH * EMBP,
                                                 _LASTN * EMBP)])

        c00 = wid * _SLOTS

        @pl.when(c00 < ntot)
        def _():
            fire_in(c00, 0)

        def body(m, carry):
            for h in range(2):
                j = 2 * m + h
                c = wid * _SLOTS + j
                nc = c + 1

                @pl.when(jnp.logical_and(nc < ntot, j + 1 < _SLOTS))
                def _():
                    fire_in(nc, 1 - h)

                @pl.when(jnp.logical_and(c < ntot, j < _SLOTS))
                def _():
                    wait_in(c, h)
                    do_chunk(c, h)

            return carry

        lax.fori_loop(0, (_SLOTS + 1) // 2, body, 0)

    return k(table_t)


_FLAT = BATCH * SEL
_RPW = _FLAT // _NW
_GCH = 128
_NCH = _RPW // _GCH
_BPW = BATCH // _NW


def _sc_gather_reduce(t32, i3):
    mesh = plsc.VectorSubcoreMesh(core_axis_name="c", subcore_axis_name="s")

    @functools.partial(
        pl.kernel,
        mesh=mesh,
        out_type=jax.ShapeDtypeStruct((BATCH, EMBP), jnp.float32),
        scratch_types=[
            pltpu.VMEM((_NCH, _GCH), jnp.int32),
            pltpu.VMEM((_NCH, _GCH), jnp.int32),
            pltpu.VMEM((_NCH, _GCH), jnp.float32),
            pltpu.VMEM((4, _GCH, 128), jnp.float32),
            pltpu.VMEM((_BPW, EMBP), jnp.float32),
            pltpu.SemaphoreType.DMA,
            pltpu.SemaphoreType.DMA,
            pltpu.SemaphoreType.DMA,
            pltpu.SemaphoreType.DMA,
        ],
        compiler_params=pltpu.CompilerParams(needs_layout_passes=False),
    )
    def k(t32_hbm, i3_hbm, out_hbm, ibuf, qbuf, cbuf, dst, acc,
          sem0, sem1, sem2, sem3):
        wid = lax.axis_index("s") * _NC + lax.axis_index("c")
        iota = lax.iota(jnp.int32, 16)
        zero = jnp.zeros((16,), jnp.float32)
        pltpu.sync_copy(i3_hbm.at[wid], ibuf)
        sems = [sem0, sem1, sem2, sem3]

        def prep(r, carry):
            for g in range(8):
                iv = ibuf[r, pl.ds(16 * g, 16)]
                qbuf[r, pl.ds(16 * g, 16)] = iv >> 2
                cf = jnp.full((16,), 1.0, jnp.float32)
                for t in _THRESH:
                    cf = jnp.where(iv >= t, cf * 0.25, cf)
                cbuf[r, pl.ds(16 * g, 16)] = cf
            return carry

        lax.fori_loop(0, _NCH, prep, 0)

        def zacc(b, carry):
            acc[b, pl.ds(0, 16)] = zero
            acc[b, pl.ds(16, 16)] = zero
            return carry

        lax.fori_loop(0, _BPW, zacc, 0)

        def fire(c, b):
            pltpu.async_copy(t32_hbm.at[qbuf.at[c]], dst.at[b], sems[b])

        def wait(b):
            pltpu.make_async_copy(t32_hbm.at[pl.ds(0, _GCH)], dst.at[b],
                                  sems[b]).wait()

        def process(c, b):
            for g in range(8):
                iv = ibuf[c, pl.ds(16 * g, 16)]
                cv = cbuf[c, pl.ds(16 * g, 16)]
                ov = (iv & 3) << 5
                rowv = iota + (g * 16)
                bv = (iota + (c * _GCH + g * 16)) // SEL
                for d in range(EMB):
                    v = plsc.load_gather(dst.at[b], [rowv, ov + d])
                    plsc.addupdate_scatter(acc, [bv, iota * 0 + d], cv * v)

        fire(0, 0)
        fire(1, 1)
        fire(2, 2)

        def loop(kk, carry):
            c0 = 4 * kk
            for j in range(4):
                nxt = c0 + j + 3

                @pl.when(nxt < _NCH)
                def _():
                    fire(nxt, (j + 3) % 4)

                wait(j)
                process(c0 + j, j)
            return carry

        lax.fori_loop(0, _NCH // 4, loop, 0)
        wait(0)
        process(_NCH - 2, 0)
        wait(1)
        process(_NCH - 1, 1)

        def norm(b, carry):
            a0 = acc[b, pl.ds(0, 16)]
            a1 = acc[b, pl.ds(16, 16)]
            n = jnp.sum(jnp.abs(a0), axis=0) + jnp.sum(jnp.abs(a1), axis=0)
            acc[b, pl.ds(0, 16)] = a0 / n
            acc[b, pl.ds(16, 16)] = a1 / n
            return carry

        lax.fori_loop(0, _BPW, norm, 0)
        pltpu.sync_copy(acc, out_hbm.at[pl.ds(wid * _BPW, _BPW)])

    return k(t32, i3)


_JB = 512
_FB = 1024


def _colsum_body(na_ref, naj_ref, cs_ref):
    m = lax.dot_general(
        na_ref[...], naj_ref[...], (((1,), (1,)), ((), ())),
        preferred_element_type=jnp.float32,
    )
    cs_ref[...] = jnp.sum(jnp.abs(m), axis=0, keepdims=True)


def _tc_colsum(na):
    return pl.pallas_call(
        _colsum_body,
        grid=(BATCH // _JB,),
        in_specs=[
            pl.BlockSpec((BATCH, EMBP), lambda j: (0, 0)),
            pl.BlockSpec((_JB, EMBP), lambda j: (j, 0)),
        ],
        out_specs=pl.BlockSpec((1, _JB), lambda j: (0, j)),
        out_shape=jax.ShapeDtypeStruct((1, BATCH), jnp.float32),
    )(na, na)


def _final_body(nai_ref, naj_ref, cs_ref, out_ref):
    m = lax.dot_general(
        nai_ref[...], naj_ref[...], (((1,), (1,)), ((), ())),
        preferred_element_type=jnp.float32,
    )
    out_ref[...] = m / cs_ref[...]


def _tc_final(na, cs):
    return pl.pallas_call(
        _final_body,
        grid=(BATCH // _FB, BATCH // _FB),
        in_specs=[
            pl.BlockSpec((_FB, EMBP), lambda i, j: (i, 0)),
            pl.BlockSpec((_FB, EMBP), lambda i, j: (j, 0)),
            pl.BlockSpec((1, _FB), lambda i, j: (0, j)),
        ],
        out_specs=pl.BlockSpec((_FB, _FB), lambda i, j: (i, j)),
        out_shape=jax.ShapeDtypeStruct((BATCH, BATCH), jnp.float32),
    )(na, na, cs)


def kernel(nodes_table, indices, layers_arr):
    del layers_arr
    ttp = jnp.pad(nodes_table.T, ((0, 0), (0, 349568 - LEN)))
    t32flat = _sc_relayout(ttp)
    t32 = t32flat.reshape(_T32R, 128)
    i3 = indices.reshape(_NW, _NCH, _GCH)
    na = _sc_gather_reduce(t32, i3)
    cs = _tc_colsum(na)
    return _tc_final(na, cs)


# speedup vs baseline: 1.1881x; 1.1002x over previous
import functools

import jax
import jax.numpy as jnp
from jax import lax
from jax.experimental import pallas as pl
from jax.experimental.pallas import tpu as pltpu
from jax.experimental.pallas import tpu_sc as plsc

LAYERS = 10
EMB = 20
EMBP = 32
LEN = (4 ** LAYERS - 1) // 3
BATCH = 4096
SEL = 50

_THRESH = [(4 ** l - 1) // 3 for l in range(1, LAYERS)]

_NC, _NS = 2, 16
_NW = _NC * _NS

_RCH = 1024
_NFULL = LEN // _RCH
_LASTN = LEN - _NFULL * _RCH
_SLOTS = (_NFULL + _NW - 1) // _NW
_T32R = (LEN * EMBP + 127) // 128
_FLATW = _T32R * 128


def _sc_relayout(table_t):
    mesh = plsc.VectorSubcoreMesh(core_axis_name="c", subcore_axis_name="s")

    @functools.partial(
        pl.kernel,
        mesh=mesh,
        out_type=jax.ShapeDtypeStruct((_FLATW,), jnp.float32),
        scratch_types=[
            pltpu.VMEM((2, EMB, _RCH), jnp.float32),
            pltpu.VMEM((_RCH * EMBP,), jnp.float32),
            pltpu.SemaphoreType.DMA,
            pltpu.SemaphoreType.DMA,
        ],
        compiler_params=pltpu.CompilerParams(needs_layout_passes=False),
    )
    def k(tt_hbm, out_hbm, inbuf, stg, isem0, isem1):
        wid = lax.axis_index("s") * _NC + lax.axis_index("c")
        iota = lax.iota(jnp.int32, 16)
        i32 = iota * EMBP
        isems = [isem0, isem1]
        ntot = _NFULL + 1

        def fire_in(c, h):
            @pl.when(c < _NFULL)
            def _():
                pltpu.async_copy(tt_hbm.at[:, pl.ds(c * _RCH, _RCH)],
                                 inbuf.at[h].at[:, pl.ds(0, _RCH)], isems[h])

            @pl.when(c == _NFULL)
            def _():
                pltpu.async_copy(tt_hbm.at[:, pl.ds(_NFULL * _RCH, 384)],
                                 inbuf.at[h].at[:, pl.ds(0, 384)], isems[h])

        def wait_in(c, h):
            @pl.when(c < _NFULL)
            def _():
                pltpu.make_async_copy(tt_hbm.at[:, pl.ds(0, _RCH)],
                                      inbuf.at[h].at[:, pl.ds(0, _RCH)],
                                      isems[h]).wait()

            @pl.when(c == _NFULL)
            def _():
                pltpu.make_async_copy(tt_hbm.at[:, pl.ds(0, 384)],
                                      inbuf.at[h].at[:, pl.ds(0, 384)],
                                      isems[h]).wait()

        def do_chunk(c, h):
            nv = jnp.where(c == _NFULL, _LASTN, _RCH)
            nvs = jnp.zeros((16,), jnp.int32) + nv
            for g in range(_RCH // 16):
                base = i32 + (g * 16 * EMBP)
                msk = (iota + (g * 16)) < nvs
                for d in range(EMB):
                    v = inbuf[h, d, pl.ds(16 * g, 16)]
                    plsc.store_scatter(stg, [base + d], v, mask=msk)
            @pl.when(c < _NFULL)
            def _():
                pltpu.sync_copy(stg,
                                out_hbm.at[pl.ds(c * _RCH * EMBP, _RCH * EMBP)])

            @pl.when(c == _NFULL)
            def _():
                pltpu.sync_copy(stg.at[pl.ds(0, _LASTN * EMBP)],
                                out_hbm.at[pl.ds(_NFULL * _RCH * EMBP,
                                                 _LASTN * EMBP)])

        c00 = wid * _SLOTS

        @pl.when(c00 < ntot)
        def _():
            fire_in(c00, 0)

        def body(m, carry):
            for h in range(2):
                j = 2 * m + h
                c = wid * _SLOTS + j
                nc = c + 1

                @pl.when(jnp.logical_and(nc < ntot, j + 1 < _SLOTS))
                def _():
                    fire_in(nc, 1 - h)

                @pl.when(jnp.logical_and(c < ntot, j < _SLOTS))
                def _():
                    wait_in(c, h)
                    do_chunk(c, h)

            return carry

        lax.fori_loop(0, (_SLOTS + 1) // 2, body, 0)

    return k(table_t)


_FLAT = BATCH * SEL
_RPW = _FLAT // _NW
_GCH = 128
_NCH = _RPW // _GCH
_BPW = BATCH // _NW


def _sc_gather_reduce(t32, i3):
    mesh = plsc.VectorSubcoreMesh(core_axis_name="c", subcore_axis_name="s")

    @functools.partial(
        pl.kernel,
        mesh=mesh,
        out_type=jax.ShapeDtypeStruct((BATCH, EMBP), jnp.float32),
        scratch_types=[
            pltpu.VMEM((_NCH, _GCH), jnp.int32),
            pltpu.VMEM((_NCH, _GCH), jnp.int32),
            pltpu.VMEM((_NCH, _GCH), jnp.float32),
            pltpu.VMEM((4, _GCH, 128), jnp.float32),
            pltpu.VMEM((_BPW, EMBP), jnp.float32),
            pltpu.SemaphoreType.DMA,
            pltpu.SemaphoreType.DMA,
            pltpu.SemaphoreType.DMA,
            pltpu.SemaphoreType.DMA,
        ],
        compiler_params=pltpu.CompilerParams(needs_layout_passes=False),
    )
    def k(t32_hbm, i3_hbm, out_hbm, ibuf, qbuf, cbuf, dst, acc,
          sem0, sem1, sem2, sem3):
        wid = lax.axis_index("s") * _NC + lax.axis_index("c")
        iota = lax.iota(jnp.int32, 16)
        zero = jnp.zeros((16,), jnp.float32)
        pltpu.sync_copy(i3_hbm.at[wid], ibuf)
        sems = [sem0, sem1, sem2, sem3]

        def prep(r, carry):
            for g in range(8):
                iv = ibuf[r, pl.ds(16 * g, 16)]
                qbuf[r, pl.ds(16 * g, 16)] = iv >> 2
                cf = jnp.full((16,), 1.0, jnp.float32)
                for t in _THRESH:
                    cf = jnp.where(iv >= t, cf * 0.25, cf)
                cbuf[r, pl.ds(16 * g, 16)] = cf
            return carry

        lax.fori_loop(0, _NCH, prep, 0)

        def zacc(b, carry):
            acc[b, pl.ds(0, 16)] = zero
            acc[b, pl.ds(16, 16)] = zero
            return carry

        lax.fori_loop(0, _BPW, zacc, 0)

        def fire(c, b):
            pltpu.async_copy(t32_hbm.at[qbuf.at[c]], dst.at[b], sems[b])

        def wait(b):
            pltpu.make_async_copy(t32_hbm.at[pl.ds(0, _GCH)], dst.at[b],
                                  sems[b]).wait()

        def process(c, b):
            for g in range(8):
                iv = ibuf[c, pl.ds(16 * g, 16)]
                cv = cbuf[c, pl.ds(16 * g, 16)]
                ov = (iv & 3) << 5
                rowv = iota + (g * 16)
                bv = iota * 8 + (c * 8 + g) // SEL
                for d in range(EMB):
                    v = plsc.load_gather(dst.at[b], [rowv, ov + d])
                    plsc.addupdate_scatter(acc, [bv, iota * 0 + d], cv * v)

        fire(0, 0)
        fire(1, 1)
        fire(2, 2)

        def loop(kk, carry):
            c0 = 4 * kk
            for j in range(4):
                nxt = c0 + j + 3

                @pl.when(nxt < _NCH)
                def _():
                    fire(nxt, (j + 3) % 4)

                wait(j)
                process(c0 + j, j)
            return carry

        lax.fori_loop(0, _NCH // 4, loop, 0)
        wait(0)
        process(_NCH - 2, 0)
        wait(1)
        process(_NCH - 1, 1)

        def norm(b, carry):
            a0 = acc[b, pl.ds(0, 16)]
            a1 = acc[b, pl.ds(16, 16)]
            n = jnp.sum(jnp.abs(a0), axis=0) + jnp.sum(jnp.abs(a1), axis=0)
            acc[b, pl.ds(0, 16)] = a0 / n
            acc[b, pl.ds(16, 16)] = a1 / n
            return carry

        lax.fori_loop(0, _BPW, norm, 0)
        pltpu.sync_copy(acc, out_hbm.at[pl.ds(wid * _BPW, _BPW)])

    return k(t32, i3)


_JB = 512
_FB = 1024


def _colsum_body(na_ref, naj_ref, cs_ref):
    m = lax.dot_general(
        na_ref[...], naj_ref[...], (((1,), (1,)), ((), ())),
        preferred_element_type=jnp.float32,
    )
    cs_ref[...] = jnp.sum(jnp.abs(m), axis=0, keepdims=True)


def _tc_colsum(na):
    return pl.pallas_call(
        _colsum_body,
        grid=(BATCH // _JB,),
        in_specs=[
            pl.BlockSpec((BATCH, EMBP), lambda j: (0, 0)),
            pl.BlockSpec((_JB, EMBP), lambda j: (j, 0)),
        ],
        out_specs=pl.BlockSpec((1, _JB), lambda j: (0, j)),
        out_shape=jax.ShapeDtypeStruct((1, BATCH), jnp.float32),
    )(na, na)


def _final_body(nai_ref, naj_ref, cs_ref, out_ref):
    m = lax.dot_general(
        nai_ref[...], naj_ref[...], (((1,), (1,)), ((), ())),
        preferred_element_type=jnp.float32,
    )
    out_ref[...] = m / cs_ref[...]


def _tc_final(na, cs):
    return pl.pallas_call(
        _final_body,
        grid=(BATCH // _FB, BATCH // _FB),
        in_specs=[
            pl.BlockSpec((_FB, EMBP), lambda i, j: (i, 0)),
            pl.BlockSpec((_FB, EMBP), lambda i, j: (j, 0)),
            pl.BlockSpec((1, _FB), lambda i, j: (0, j)),
        ],
        out_specs=pl.BlockSpec((_FB, _FB), lambda i, j: (i, j)),
        out_shape=jax.ShapeDtypeStruct((BATCH, BATCH), jnp.float32),
    )(na, na, cs)


def kernel(nodes_table, indices, layers_arr):
    del layers_arr
    ttp = jnp.pad(nodes_table.T, ((0, 0), (0, 349568 - LEN)))
    t32flat = _sc_relayout(ttp)
    t32 = t32flat.reshape(_T32R, 128)
    i3 = indices.reshape(_NW, 16, 400).transpose(0, 2, 1).reshape(_NW, _NCH, _GCH)
    na = _sc_gather_reduce(t32, i3)
    cs = _tc_colsum(na)
    return _tc_final(na, cs)
